# baseline (speedup 1.0000x reference)
import functools

import jax
import jax.numpy as jnp
from jax import lax
from jax.experimental import pallas as pl
from jax.experimental.pallas import tpu as pltpu
from jax.experimental.pallas import tpu_sc as plsc

_B, _N, _D, _H, _W, _C = 4, 1, 48, 32, 88, 64
_NX, _NY = 200, 200
_P = _N * _D * _H * _W
_HW = _H * _W
_NCORES, _NTILES = 2, 16
_PT = _P // _NTILES
_NSUB = 4
_CH = _PT // _NSUB
_NGRP = _CH // 16
_HALF = _NX // _NCORES
_LOCAL = _HALF * _NY
_ACC_ROWS = 20480
_ZCH = _ACC_ROWS // _NTILES // 128
_DUMMY = _LOCAL
_BLK = 128

_BXM = (-50.0, -50.0, -10.0)
_DX = (0.5, 0.5, 20.0)


def _body(x_hbm, geom_hbm, sem_hbm, out_hbm,
          geom_v, sem_v, pid_v, lidx_v, rows_v, zero_v, acc_sh, gsem):
    c = lax.axis_index("c")
    s = lax.axis_index("s")
    lanes = lax.iota(jnp.int32, 16)

    def zinit(i, _):
        for k in range(4):
            zero_v[i, pl.ds(k * 16, 16)] = jnp.zeros((16,), jnp.float32)
        return 0
    lax.fori_loop(0, 128, zinit, 0)

    for b in range(_B):
        def zacc(i, _):
            pltpu.sync_copy(zero_v, acc_sh.at[pl.ds(s * 1280 + i * 128, 128)])
            return 0
        lax.fori_loop(0, _ZCH, zacc, 0)
        pltpu.sync_copy(sem_hbm.at[pl.ds(b * _HW, _HW)], sem_v)
        plsc.subcore_barrier()

        for sub in range(_NSUB):
            p0 = s * _PT + sub * _CH
            pltpu.sync_copy(
                geom_hbm.at[pl.ds((b * _P + p0) * 3, _CH * 3)], geom_v)

            def grp(g, cnt):
                pin = p0 + g * 16 + lanes
                hw = lax.rem(pin, _HW)
                semv = plsc.load_gather(sem_v, [hw])
                j3 = (g * 16 + lanes) * 3
                px = plsc.load_gather(geom_v, [j3])
                py = plsc.load_gather(geom_v, [j3 + 1])
                pz = plsc.load_gather(geom_v, [j3 + 2])
                gx = ((px - _BXM[0]) / _DX[0]).astype(jnp.int32)
                gy = ((py - _BXM[1]) / _DX[1]).astype(jnp.int32)
                gz = ((pz - _BXM[2]) / _DX[2]).astype(jnp.int32)
                gxl = gx - c * _HALF
                kept = ((gxl >= 0) & (gxl < _HALF)
                        & (gy >= 0) & (gy < _NY)
                        & (gz >= 0) & (gz < 1)
                        & (semv > 0.5))
                lidx = gxl * _NY + gy
                pid = b * _P + pin
                plsc.store_compressed(pid_v.at[pl.ds(cnt, 16)], pid, mask=kept)
                plsc.store_compressed(lidx_v.at[pl.ds(cnt, 16)], lidx,
                                      mask=kept)
                return cnt + jnp.sum(kept.astype(jnp.int32))

            cnt = lax.fori_loop(0, _NGRP, grp, jnp.int32(0))

            for k in range(_BLK // 16):
                pid_v[pl.ds(cnt + k * 16, 16)] = jnp.zeros((16,), jnp.int32)
                lidx_v[pl.ds(cnt + k * 16, 16)] = jnp.full(
                    (16,), _DUMMY, jnp.int32)

            nblk = (cnt + _BLK - 1) // _BLK

            def blk(i, _):
                pltpu.async_copy(
                    x_hbm.at[pid_v.at[pl.ds(i * _BLK, _BLK)]],
                    rows_v, gsem).wait()
                for k in range(_BLK // 16):
                    lv = lidx_v[pl.ds(i * _BLK + k * 16, 16)]
                    pltpu.sync_copy(rows_v.at[pl.ds(k * 16, 16)],
                                    acc_sh.at[lv], add=True)
                return 0

            lax.fori_loop(0, nblk, blk, 0)

        plsc.subcore_barrier()
        base = b * (_NX * _NY) + c * _LOCAL

        def cpout(i, _):
            xr = i * _NTILES + s
            pltpu.sync_copy(acc_sh.at[pl.ds(xr * _NY, _NY)],
                            out_hbm.at[pl.ds(base + xr * _NY, _NY)])
            return 0

        lax.fori_loop(0, (_HALF - s + _NTILES - 1) // _NTILES, cpout, 0)
        plsc.subcore_barrier()


@functools.partial(
    pl.kernel,
    out_type=jax.ShapeDtypeStruct((_B * _NX * _NY, _C), jnp.float32),
    mesh=plsc.VectorSubcoreMesh(core_axis_name="c", subcore_axis_name="s"),
    compiler_params=pltpu.CompilerParams(
        needs_layout_passes=False, use_tc_tiling_on_sc=False),
    scratch_types=[
        pltpu.VMEM((_CH * 3,), jnp.float32),
        pltpu.VMEM((_HW,), jnp.float32),
        pltpu.VMEM((_CH + _BLK,), jnp.int32),
        pltpu.VMEM((_CH + _BLK,), jnp.int32),
        pltpu.VMEM((_BLK, _C), jnp.float32),
        pltpu.VMEM((128, _C), jnp.float32),
        pltpu.VMEM_SHARED((_ACC_ROWS, _C), jnp.float32),
        pltpu.SemaphoreType.DMA,
    ],
)
def _bev_pool(x_hbm, geom_hbm, sem_hbm, out_hbm, *scratch):
    _body(x_hbm, geom_hbm, sem_hbm, out_hbm, *scratch)


def kernel(x, geom_feats, semantic_mask):
    xf = x.reshape(_B * _P, _C)
    geomf = geom_feats.reshape(_B * _P * 3)
    semf = semantic_mask[:, 1].reshape(_B * _HW)
    flat = _bev_pool(xf, geomf, semf)
    return (flat.reshape(_B, _NX, _NY, _C)
                .transpose(0, 3, 1, 2)
                .reshape(_B, _N * _C, _NX, _NY))


# speedup vs baseline: 1.9232x; 1.9232x over previous
import functools

import jax
import jax.numpy as jnp
from jax import lax
from jax.experimental import pallas as pl
from jax.experimental.pallas import tpu as pltpu
from jax.experimental.pallas import tpu_sc as plsc

_B, _D, _H, _W, _C = 4, 48, 32, 88, 64
_NX, _NY = 200, 200
_HW = _H * _W
_NCORES, _NTILES = 2, 16
_SLABS = _D * _H
_ST = _SLABS // _NTILES
_SG = 2
_PERD = _H // _SG
_NSUB = _ST // _SG
_PSUB = _SG * _W
_FLEN = _SG * _C * _W
_HALF = _NX // _NCORES
_LOCAL = _HALF * _NY
_ACC_ROWS = 20096
_ZB = 32
_NZB = _ACC_ROWS // _ZB
_DUMMY = _LOCAL
_BLK = 64
_NGRP = 6


def _body(x_hbm, geom_hbm, sem_hbm, out_hbm,
          feat_v, gx_v, gy_v, gz_v, sem_v, eb_v, lidx_v, stag_a, stag_b,
          src_v, xp_v, zero_v, acc_sh, fsem, ssem):
    ci = lax.axis_index("c")
    s = lax.axis_index("s")
    lanes = lax.iota(jnp.int32, 16)
    c100 = ci * _HALF
    stags = (stag_a, stag_b)

    def zinit(i, _):
        for k in range(4):
            zero_v[i, pl.ds(k * 16, 16)] = jnp.zeros((16,), jnp.float32)
        return 0
    lax.fori_loop(0, _ZB, zinit, 0)

    def extract_block(blk, stag2d):
        for k in range(_BLK // 16):
            eb = eb_v[pl.ds(blk * _BLK + k * 16, 16)]
            rows = lanes + k * 16

            def xt(c8, _):
                for dc in range(8):
                    cch = c8 * 8 + dc
                    vals = plsc.load_gather(feat_v, [eb + cch * _W])
                    plsc.store_scatter(stag2d, [rows, cch + (lanes * 0)],
                                       vals)
                return 0
            lax.fori_loop(0, _C // 8, xt, 0)

    def scatter_block(blk, stag2d):
        for k in range(_BLK // 16):
            lv = lidx_v[pl.ds(blk * _BLK + k * 16, 16)]
            pltpu.async_copy(stag2d.at[pl.ds(k * 16, 16)], acc_sh.at[lv],
                             ssem, add=True)

    def drain_block(stag2d):
        for k in range(_BLK // 16):
            pltpu.make_async_copy(stag2d.at[pl.ds(k * 16, 16)],
                                  acc_sh.at[pl.ds(0, 16)], ssem).wait()

    def run_batch(b, _):
        def zacc(i, _):
            blkid = i * _NTILES + s
            pltpu.sync_copy(zero_v, acc_sh.at[pl.ds(blkid * _ZB, _ZB)])
            return 0
        lax.fori_loop(0, (_NZB - s + _NTILES - 1) // _NTILES, zacc, 0)
        plsc.subcore_barrier()

        def sub(j, _):
            dp = j // _PERD
            h0 = (j % _PERD) * _SG
            slab0 = (b * _SLABS + s * _ST + dp * _H + h0) * _C * _W
            fcp = pltpu.async_copy(
                x_hbm.at[pl.ds(slab0, _FLEN)],
                feat_v.at[pl.ds(0, _FLEN)], fsem)
            d = s * 3 + dp
            gbase = ((b * _D + d) * 3) * _HW + h0 * _W
            pltpu.sync_copy(geom_hbm.at[pl.ds(gbase, _PSUB)],
                            gx_v.at[pl.ds(0, _PSUB)])
            pltpu.sync_copy(geom_hbm.at[pl.ds(gbase + _HW, _PSUB)],
                            gy_v.at[pl.ds(0, _PSUB)])
            pltpu.sync_copy(geom_hbm.at[pl.ds(gbase + 2 * _HW, _PSUB)],
                            gz_v.at[pl.ds(0, _PSUB)])
            pltpu.sync_copy(sem_hbm.at[pl.ds(b * _HW + h0 * _W, _PSUB)],
                            sem_v.at[pl.ds(0, _PSUB)])

            cnt = jnp.int32(0)
            for hl in range(_SG):
                def grp(g, cnt):
                    off = hl * _W + g * 16
                    w = g * 16 + lanes
                    px = gx_v[pl.ds(off, 16)]
                    py = gy_v[pl.ds(off, 16)]
                    pz = gz_v[pl.ds(off, 16)]
                    semv = sem_v[pl.ds(off, 16)]
                    gx = ((px + 50.0) * 2.0).astype(jnp.int32)
                    gy = ((py + 50.0) * 2.0).astype(jnp.int32)
                    tz = pz + 10.0
                    gxl = gx - c100
                    kept = ((w < _W)
                            & (gxl >= 0) & (gxl < _HALF)
                            & (gy >= 0) & (gy < _NY)
                            & (tz > -20.0) & (tz < 20.0)
                            & (semv > 0.5))
                    lidx = gxl * _NY + gy
                    eb = w + hl * (_C * _W)
                    plsc.store_compressed(eb_v.at[pl.ds(cnt, 16)], eb,
                                          mask=kept)
                    plsc.store_compressed(lidx_v.at[pl.ds(cnt, 16)], lidx,
                                          mask=kept)
                    return cnt + plsc.all_reduce_population_count(kept)[0]

                cnt = lax.fori_loop(0, _NGRP, grp, cnt)

            for k in range(_BLK // 16):
                eb_v[pl.ds(cnt + k * 16, 16)] = jnp.zeros((16,), jnp.int32)
                lidx_v[pl.ds(cnt + k * 16, 16)] = jnp.full(
                    (16,), _DUMMY, jnp.int32)

            nblk = (cnt + _BLK - 1) // _BLK
            fcp.wait()

            def blk(i, _):
                for par in range(2):
                    @pl.when(lax.rem(i, 2) == par)
                    def _():
                        stag2d = stags[par]
                        @pl.when(i >= 2)
                        def _():
                            drain_block(stag2d)
                        extract_block(i, stag2d)
                        scatter_block(i, stag2d)
                return 0

            lax.fori_loop(0, nblk, blk, 0)
            def drain(i, _):
                for par in range(2):
                    @pl.when(lax.rem(i, 2) == par)
                    def _():
                        drain_block(stags[par])
                return 0
            lax.fori_loop(jnp.maximum(nblk - 2, 0), nblk, drain, 0)
            return 0

        lax.fori_loop(0, _NSUB, sub, 0)

        plsc.subcore_barrier()
        def cpout(i, _):
            xr = i * _NTILES + s
            for y0, ylen in ((0, 96), (96, 104)):
                pltpu.sync_copy(acc_sh.at[pl.ds(xr * _NY + y0, ylen)],
                                src_v.at[pl.ds(0, ylen)])

                def xt(cch, _):
                    for m in range(-(-ylen // 16)):
                        vals = plsc.load_gather(
                            src_v, [m * 16 + lanes, cch + (lanes * 0)])
                        xp_v[cch, 0, pl.ds(m * 16, 16)] = vals
                    return 0
                lax.fori_loop(0, _C, xt, 0)
                pltpu.sync_copy(
                    xp_v.at[:, :, pl.ds(0, ylen)],
                    out_hbm.at[pl.ds(b * _C, _C), pl.ds(c100 + xr, 1),
                               pl.ds(y0, ylen)])
            return 0

        lax.fori_loop(0, (_HALF - s + _NTILES - 1) // _NTILES, cpout, 0)
        plsc.subcore_barrier()
        return 0

    lax.fori_loop(0, _B, run_batch, 0)


@functools.partial(
    pl.kernel,
    out_type=jax.ShapeDtypeStruct((_B * _C, _NX, _NY), jnp.float32),
    mesh=plsc.VectorSubcoreMesh(core_axis_name="c", subcore_axis_name="s"),
    compiler_params=pltpu.CompilerParams(
        needs_layout_passes=False, use_tc_tiling_on_sc=False),
    scratch_types=[
        pltpu.VMEM((_FLEN + 16,), jnp.float32),
        pltpu.VMEM((_PSUB + 16,), jnp.float32),
        pltpu.VMEM((_PSUB + 16,), jnp.float32),
        pltpu.VMEM((_PSUB + 16,), jnp.float32),
        pltpu.VMEM((_PSUB + 16,), jnp.float32),
        pltpu.VMEM((_PSUB + _BLK,), jnp.int32),
        pltpu.VMEM((_PSUB + _BLK,), jnp.int32),
        pltpu.VMEM((_BLK, _C), jnp.float32),
        pltpu.VMEM((_BLK, _C), jnp.float32),
        pltpu.VMEM((112, _C), jnp.float32),
        pltpu.VMEM((_C, 1, 120), jnp.float32),
        pltpu.VMEM((_ZB, _C), jnp.float32),
        pltpu.VMEM_SHARED((_ACC_ROWS, _C), jnp.float32),
        pltpu.SemaphoreType.DMA,
        pltpu.SemaphoreType.DMA,
    ],
)
def _bev_pool(x_hbm, geom_hbm, sem_hbm, out_hbm, *scratch):
    _body(x_hbm, geom_hbm, sem_hbm, out_hbm, *scratch)


def kernel(x, geom_feats, semantic_mask):
    xw = jnp.transpose(x, (0, 1, 2, 3, 5, 4)).reshape(_B * _D * _H * _C * _W)
    geomp = jnp.transpose(geom_feats, (0, 1, 2, 5, 3, 4)).reshape(
        _B * _D * 3 * _HW)
    semf = semantic_mask[:, 1].reshape(_B * _HW)
    out = _bev_pool(xw, geomp, semf)
    return out.reshape(_B, _C, _NX, _NY)
